# Initial kernel scaffold; baseline (speedup 1.0000x reference)
import jax
import jax.numpy as jnp
from jax.experimental import pallas as pl


def kernel(heatmap):
    raise NotImplementedError("write your pallas kernel here")


# speedup vs baseline: 12.4425x; 12.4425x over previous
import jax
import jax.numpy as jnp
from jax.experimental import pallas as pl
from jax.experimental.pallas import tpu as pltpu

_N_TARGETS = 6
_R = 5
_H = 512
_W = 512
_AH = _H - _R + 1
_HP = _H + 32


def _iota(shape, dim):
    return jax.lax.broadcasted_iota(jnp.int32, shape, dim)


def _hsum(t):
    acc = t
    n = t.shape[0]
    for k in range(1, _R):
        acc = acc + jnp.concatenate(
            [t[:, k:], jnp.zeros((n, k), jnp.float32)], axis=1)
    return acc


def _rowstats(aggblk, colia):
    m = jnp.max(aggblk, axis=1, keepdims=True)
    cc = jnp.min(jnp.where(aggblk == m, colia, jnp.int32(_W)),
                 axis=1, keepdims=True)
    return m, cc


def _peaks_kernel(hm_ref, out_ref, hm_s, h_s, rmax_s, rcol_s):
    hm0 = hm_ref[0]
    hm_s[...] = hm0

    h0 = _hsum(hm0)
    h_s[0:_H, :] = h0
    h_s[_H:_HP, :] = jnp.zeros((_HP - _H, _W), jnp.float32)

    v = h0[0:_AH, :]
    for k in range(1, _R):
        v = v + h0[k:k + _AH, :]
    colia = _iota((_AH, _W), 1)
    agg0 = jnp.where(colia < _AH, v / float(_R * _R), -1.0)
    m0, c0 = _rowstats(agg0, colia)
    rmax_s[0:_AH, :] = m0
    rmax_s[_AH:_H, :] = jnp.full((_H - _AH, 1), -1.0, jnp.float32)
    rcol_s[0:_AH, :] = c0
    rcol_s[_AH:_H, :] = jnp.zeros((_H - _AH, 1), jnp.int32)

    flat = _iota((_H, _W), 0) * _W + _iota((_H, _W), 1)
    big = jnp.int32(_H * _W)
    rowi16 = _iota((16, _W), 0)
    coli16 = _iota((16, _W), 1)
    rio = _iota((_H, 1), 0)

    def body(i, res):
        rm = rmax_s[...]
        gmax = jnp.max(rm)
        r = jnp.min(jnp.where(rm == gmax, rio, jnp.int32(_H)))
        c = jnp.min(jnp.where(rio == r, rcol_s[...], jnp.int32(_W)))

        rs = pl.multiple_of(jnp.minimum((r // 8) * 8, _H - 16), 8)
        tile = hm_s[pl.ds(rs, 16), :]
        inwin = ((rowi16 >= r - rs) & (rowi16 < r - rs + _R)
                 & (coli16 >= c) & (coli16 < c + _R))
        mval = jnp.max(jnp.where(inwin, tile, -1.0))
        conf = jnp.sum(jnp.where(inwin, tile, 0.0))

        hm = hm_s[...]
        fi2 = jnp.min(jnp.where(hm == mval, flat, big))
        rh = fi2 // _W
        ch = jax.lax.rem(fi2, _W)

        ztile = jnp.where(inwin, 0.0, tile)
        hm_s[pl.ds(rs, 16), :] = ztile
        h_s[pl.ds(rs, 16), :] = _hsum(ztile)

        rs3 = pl.multiple_of(
            jnp.minimum(jnp.maximum(((r - (_R - 1)) // 8) * 8, 0), _H - 16), 8)
        h32 = h_s[pl.ds(rs3, 32), :]
        acc = h32[0:16]
        for k in range(1, _R):
            acc = acc + h32[k:k + 16]
        rowabs = rs3 + rowi16
        aggblk = jnp.where((rowabs < _AH) & (coli16 < _AH),
                           acc / float(_R * _R), -1.0)
        mb, cb = _rowstats(aggblk, coli16)
        rmax_s[pl.ds(rs3, 16), :] = mb
        rcol_s[pl.ds(rs3, 16), :] = cb

        sel = _iota((8, 128), 0) == i
        coli8 = _iota((8, 128), 1)
        res = jnp.where(sel & (coli8 == 0), rh.astype(jnp.float32), res)
        res = jnp.where(sel & (coli8 == 1), ch.astype(jnp.float32), res)
        res = jnp.where(sel & (coli8 == 2), conf, res)
        return res

    res = jax.lax.fori_loop(0, _N_TARGETS, body,
                            jnp.zeros((8, 128), jnp.float32))
    out_ref[0] = res


def kernel(heatmap):
    hm = heatmap[:, 0]
    b = hm.shape[0]
    out = pl.pallas_call(
        _peaks_kernel,
        grid=(b,),
        in_specs=[pl.BlockSpec((1, _H, _W), lambda i: (i, 0, 0))],
        out_specs=pl.BlockSpec((1, 8, 128), lambda i: (i, 0, 0)),
        out_shape=jax.ShapeDtypeStruct((b, 8, 128), jnp.float32),
        scratch_shapes=[
            pltpu.VMEM((_H, _W), jnp.float32),
            pltpu.VMEM((_HP, _W), jnp.float32),
            pltpu.VMEM((_H, 1), jnp.float32),
            pltpu.VMEM((_H, 1), jnp.int32),
        ],
        compiler_params=pltpu.CompilerParams(
            dimension_semantics=("parallel",),
        ),
    )(hm)
    end_points = out[:, :_N_TARGETS, 0:2]
    confidences = out[:, :_N_TARGETS, 2]
    return end_points, confidences
